# Initial kernel scaffold; baseline (speedup 1.0000x reference)
import jax
import jax.numpy as jnp
from jax.experimental import pallas as pl


def kernel(prediction, classes_present):
    raise NotImplementedError("write your pallas kernel here")


# speedup vs baseline: 1.2105x; 1.2105x over previous
import functools

import jax
import jax.numpy as jnp
from jax import lax
from jax.experimental import pallas as pl
from jax.experimental.pallas import tpu as pltpu
from jax.experimental.pallas import tpu_sc as plsc

_CONF = 0.25
_NMS = 0.45
_NCLS = 80
_IMG = 640.0
_N = 5000
_NP = 5120
_B = 2
_TJ = 512
_TI = 512
_F = 16
_VF = 128

_NW = 32
_ROWS = _B * _NP
_RPW = _ROWS // _NW
_CH = 64
_NCH = _RPW // _CH


def _prep_body(pred_ref, cp_ref, feat_ref):
    x = pred_ref[...]
    cp = cp_ref[...]
    cxs = x[:, 0:1] * _IMG
    cys = x[:, 1:2] * _IMG
    ws = x[:, 2:3] * _IMG
    hs = x[:, 3:4] * _IMG
    x1 = cxs - ws / 2.0
    y1 = cys - hs / 2.0
    x2 = cxs + ws / 2.0
    y2 = cys + hs / 2.0
    li = lax.broadcasted_iota(jnp.int32, x.shape, 1)
    valid = (li >= 5) & (li < 5 + _NCLS)
    pm = jnp.where(valid, x, -jnp.inf)
    cls_conf = jnp.max(pm, axis=1, keepdims=True)
    idl = jnp.where(valid & (x == cls_conf), li - 5, 2**30)
    cls_id = jnp.min(idl, axis=1, keepdims=True)
    cls_f = cls_id.astype(jnp.float32)
    score = x[:, 4:5] * cls_conf
    maskf = jnp.where(score > _CONF, 1.0, 0.0)
    presf = jnp.max(jnp.where(cls_id == cp, 1.0, 0.0), axis=1, keepdims=True)
    off = cls_f * (2.0 * _IMG)
    zeros4 = jnp.zeros_like(x[:, 0:4])
    feat = jnp.concatenate(
        [x1 + off, y1 + off, x2 + off, y2 + off,
         score, cls_f, maskf, presf,
         x1, y1, x2, y2, zeros4], axis=1)
    feat_ref[...] = feat


def _pair_body(featc_ref, featr_ref, val_ref, idx_ref):
    b = pl.program_id(0)
    jt = pl.program_id(1)
    fc = featc_ref[...]
    x1j = fc[:, 0:1]
    y1j = fc[:, 1:2]
    x2j = fc[:, 2:3]
    y2j = fc[:, 3:4]
    sj = fc[:, 4:5]
    gj = jt * _TJ + lax.broadcasted_iota(jnp.int32, (_TJ, 1), 0)
    areaj = (x2j - x1j) * (y2j - y1j)
    supp = jnp.zeros((_TJ, 1), jnp.float32)
    rank = jnp.zeros((_TJ, 1), jnp.float32)
    for k in range(_NP // _TI):
        i0 = k * _TI
        fr = featr_ref[:, i0:i0 + _TI]
        x1i = fr[0:1, :]
        y1i = fr[1:2, :]
        x2i = fr[2:3, :]
        y2i = fr[3:4, :]
        si = fr[4:5, :]
        mi = fr[6:7, :]
        gi = i0 + lax.broadcasted_iota(jnp.int32, (1, _TI), 1)
        dom = (si > sj) | ((si == sj) & (gi < gj))
        xx1 = jnp.maximum(x1i, x1j)
        yy1 = jnp.maximum(y1i, y1j)
        xx2 = jnp.minimum(x2i, x2j)
        yy2 = jnp.minimum(y2i, y2j)
        inter = jnp.clip(xx2 - xx1, 0.0) * jnp.clip(yy2 - yy1, 0.0)
        areai = (x2i - x1i) * (y2i - y1i)
        union = areai + areaj - inter
        iou = inter / jnp.maximum(union, 1e-9)
        hit = dom & (mi > 0.5) & (iou > _NMS)
        supp = jnp.maximum(
            supp, jnp.max(jnp.where(hit, 1.0, 0.0), axis=1, keepdims=True))
        rank = rank + jnp.sum(jnp.where(dom, 1.0, 0.0), axis=1, keepdims=True)
    keep = (fc[:, 6:7] > 0.5) & (supp < 0.5) & (fc[:, 7:8] > 0.5)
    finalf = jnp.where(keep, 1.0, 0.0)
    zpad = jnp.zeros((_TJ, _VF - 6), jnp.float32)
    val = jnp.concatenate(
        [fc[:, 8:12] * finalf, fc[:, 4:5] * finalf, fc[:, 5:6] * finalf,
         zpad], axis=1)
    val_ref[...] = val
    idx_ref[...] = b * _NP + rank.astype(jnp.int32)


def _sc_scatter_body(val_hbm, idx_hbm, out_hbm, idx_v, rows_v, sem):
    wid = lax.axis_index("s") * 2 + lax.axis_index("c")
    base = wid * _RPW
    pltpu.sync_copy(idx_hbm.at[wid], idx_v)
    pltpu.sync_copy(val_hbm.at[pl.ds(base, _RPW)], rows_v)
    copies = [
        pltpu.async_copy(rows_v.at[pl.ds(c * _CH, _CH)],
                         out_hbm.at[idx_v.at[c]], sem)
        for c in range(_NCH)
    ]
    for c_ in copies:
        c_.wait()


@jax.jit
def kernel(prediction, classes_present):
    pred = jnp.pad(prediction.astype(jnp.float32),
                   ((0, 0), (0, _NP - _N), (0, 128 - prediction.shape[-1])))
    cp = jnp.pad(classes_present.reshape(1, -1).astype(jnp.int32),
                 ((0, 0), (0, 128 - classes_present.shape[0])),
                 constant_values=-1)

    feat = pl.pallas_call(
        _prep_body,
        grid=(_B, _NP // _TJ),
        in_specs=[
            pl.BlockSpec((None, _TJ, 128), lambda b, j: (b, j, 0)),
            pl.BlockSpec((1, 128), lambda b, j: (0, 0)),
        ],
        out_specs=pl.BlockSpec((None, _TJ, _F), lambda b, j: (b, j, 0)),
        out_shape=jax.ShapeDtypeStruct((_B, _NP, _F), jnp.float32),
    )(pred, cp)

    featr = jnp.swapaxes(feat, 1, 2)

    val, idx = pl.pallas_call(
        _pair_body,
        grid=(_B, _NP // _TJ),
        in_specs=[
            pl.BlockSpec((None, _TJ, _F), lambda b, j: (b, j, 0)),
            pl.BlockSpec((None, _F, _NP), lambda b, j: (b, 0, 0)),
        ],
        out_specs=[
            pl.BlockSpec((None, _TJ, _VF), lambda b, j: (b, j, 0)),
            pl.BlockSpec((None, _TJ, 1), lambda b, j: (b, j, 0)),
        ],
        out_shape=[
            jax.ShapeDtypeStruct((_B, _NP, _VF), jnp.float32),
            jax.ShapeDtypeStruct((_B, _NP, 1), jnp.int32),
        ],
    )(feat, featr)

    val_flat = val.reshape(_ROWS, _VF)
    idx_flat = idx.reshape(_NW, _NCH, _CH)

    scatter = functools.partial(
        pl.kernel,
        mesh=plsc.VectorSubcoreMesh(core_axis_name="c", subcore_axis_name="s"),
        out_type=jax.ShapeDtypeStruct((_ROWS, _VF), jnp.float32),
        scratch_types=[
            pltpu.VMEM((_NCH, _CH), jnp.int32),
            pltpu.VMEM((_RPW, _VF), jnp.float32),
            pltpu.SemaphoreType.DMA,
        ],
    )(_sc_scatter_body)
    out = scatter(val_flat, idx_flat)

    return out.reshape(_B, _NP, _VF)[:, :_N, :6]
